# baseline (speedup 1.0000x reference)
import functools

import jax
import jax.numpy as jnp
from jax import lax
from jax.experimental import pallas as pl
from jax.experimental.pallas import tpu as pltpu
from jax.experimental.pallas import tpu_sc as plsc

N = 10000
E = 320000
D = 128
H = 64
NT = 16
ET = E // NT
C = 32
NCHUNK = ET // C


def _node_pre_body(x_ref, wsg, wdg, wdu, wsu, bdu, bsu,
                   tsrc0, tsrc1, edst, cx):
    x = x_ref[...]
    t_es = x @ wsg[...]
    t_bh = x @ wdu[...] + bdu[...]
    tsrc0[...] = jnp.concatenate([t_es[:, :H], t_bh[:, :H]], axis=1)
    tsrc1[...] = jnp.concatenate([t_es[:, H:], t_bh[:, H:]], axis=1)
    edst[...] = x @ wdg[...]
    cx[...] = x @ wsu[...] + bsu[...]


def _gate_body(ef, weg, bias, eg0, eg1):
    t = ef[...] @ weg[...] + bias[...]
    eg0[...] = t[:, :H]
    eg1[...] = t[:, H:]


def _sc_body(src_hbm, dst_hbm, ts0, ts1, td, eg0, eg1, zer,
             m0_hbm, m1_hbm, acc0_hbm, acc1_hbm, stats_hbm,
             sq0, sq1, tq0, tq1, av0, av1, dv0, dv1, ev0, ev1, cb0, cb1,
             lo0, lo1, stat_v, acc_sh,
             ss0, ss1, st0, st1, sa0, sa1, sd0, sd1, se0, se1,
             sm0, sm1, sx0, sx1):
    c = lax.axis_index("c")
    s = lax.axis_index("s")
    srcq = [sq0, sq1]
    dstq = [tq0, tq1]
    av = [av0, av1]
    dvv = [dv0, dv1]
    ev = [ev0, ev1]
    cbv = [cb0, cb1]
    lov = [lo0, lo1]
    sem_s = [ss0, ss1]
    sem_t = [st0, st1]
    sem_a = [sa0, sa1]
    sem_d = [sd0, sd1]
    sem_e = [se0, se1]
    sem_m = [sm0, sm1]
    sem_x = [sx0, sx1]

    @pl.when(s == 0)
    def _():
        pltpu.sync_copy(zer, acc_sh)

    plsc.subcore_barrier()

    def run_half(ts, eg, off, m_hbm):
        tile_base = s * ET

        def cbase(ci):
            return tile_base + jnp.minimum(ci * C, ET - C)

        def idx_issue(ci, q):
            b = cbase(ci)
            pltpu.async_copy(src_hbm.at[pl.ds(b, C)], srcq[q], sem_s[q])
            pltpu.async_copy(dst_hbm.at[pl.ds(b, C)], dstq[q], sem_t[q])

        def idx_wait(q):
            pltpu.make_async_copy(src_hbm.at[pl.ds(0, C)], srcq[q],
                                  sem_s[q]).wait()
            pltpu.make_async_copy(dst_hbm.at[pl.ds(0, C)], dstq[q],
                                  sem_t[q]).wait()

        def gather_issue(ci, g, q):
            b = cbase(ci)
            pltpu.async_copy(ts.at[srcq[q]], av[g], sem_a[g])
            pltpu.async_copy(td.at[dstq[q]], dvv[g], sem_d[g])
            pltpu.async_copy(eg.at[pl.ds(b, C)], ev[g], sem_e[g])

        def gather_wait(g, q):
            pltpu.make_async_copy(ts.at[srcq[q]], av[g], sem_a[g]).wait()
            pltpu.make_async_copy(td.at[dstq[q]], dvv[g], sem_d[g]).wait()
            pltpu.make_async_copy(eg.at[pl.ds(0, C)], ev[g], sem_e[g]).wait()

        def ilo_compute(g, q):
            for j in range(C // 16):
                sl = pl.ds(j * 16, 16)
                lov[g][sl] = dstq[q][sl]

        def rows(g, carry):
            def row_body(r2, rc):
                acc = list(rc)
                for p in range(2):
                    r = r2 * 2 + p
                    for k in range(4):
                        sl = pl.ds(k * 16, 16)
                        slh = pl.ds(off + k * 16, 16)
                        m = av[g][r, sl] + dvv[g][r, slh] + ev[g][r, sl]
                        ev[g][r, sl] = m
                        sig = 1.0 / (1.0 + jnp.exp(-m))
                        cbv[g][r, sl] = sig * av[g][r, pl.ds(H + k * 16, 16)]
                        cbv[g][r, pl.ds(H + k * 16, 16)] = sig
                        acc[k] = acc[k] + m
                        acc[4 + k] = acc[4 + k] + m * m
                return tuple(acc)

            return lax.fori_loop(0, C // 2, row_body, carry)

        def outs_issue(ci, g):
            b = cbase(ci)
            pltpu.async_copy(ev[g], m_hbm.at[pl.ds(b, C)], sem_m[g])
            pltpu.async_copy(cbv[g], acc_sh.at[lov[g]], sem_x[g], add=True)

        def outs_wait(g):
            pltpu.make_async_copy(ev[g], m_hbm.at[pl.ds(0, C)],
                                  sem_m[g]).wait()
            pltpu.make_async_copy(cbv[g], acc_sh.at[lov[g]], sem_x[g]).wait()

        idx_issue(0, 0)
        idx_wait(0)
        gather_issue(0, 1, 0)
        ilo_compute(1, 0)
        idx_issue(1, 1)
        gather_wait(1, 0)
        idx_issue(2, 0)
        z = jnp.zeros((16,), jnp.float32)
        carry = rows(1, (z,) * 8)
        outs_issue(0, 1)
        idx_wait(1)
        gather_issue(1, 0, 1)
        ilo_compute(0, 1)

        def pair_body(i2, carry):
            i = 1 + 2 * i2
            gather_wait(0, 1)
            idx_issue(i + 2, 1)
            carry = rows(0, carry)
            outs_issue(i, 0)
            outs_wait(1)
            idx_wait(0)
            gather_issue(i + 1, 1, 0)
            ilo_compute(1, 0)
            gather_wait(1, 0)
            idx_issue(i + 3, 0)
            carry = rows(1, carry)
            outs_issue(i + 1, 1)
            outs_wait(0)
            idx_wait(1)
            gather_issue(i + 2, 0, 1)
            ilo_compute(0, 1)
            return carry

        carry = lax.fori_loop(0, (NCHUNK - 1) // 2, pair_body, carry)

        outs_wait(1)
        gather_wait(0, 1)
        idx_wait(0)

        for k in range(4):
            stat_v[pl.ds(k * 16, 16)] = carry[k]
            stat_v[pl.ds(H + k * 16, 16)] = carry[4 + k]
        pltpu.sync_copy(stat_v, stats_hbm.at[c, s])

    @pl.when(c == 0)
    def _():
        run_half(ts0, eg0, 0, m0_hbm)

    @pl.when(c == 1)
    def _():
        run_half(ts1, eg1, H, m1_hbm)

    plsc.subcore_barrier()

    @pl.when((s == 0) & (c == 0))
    def _():
        pltpu.sync_copy(acc_sh, acc0_hbm)

    @pl.when((s == 0) & (c == 1))
    def _():
        pltpu.sync_copy(acc_sh, acc1_hbm)


def _edge_epi_body(m0, m1, ef, stats, gamma, beta, y):
    st = stats[...]
    red = jnp.sum(st, axis=1)
    sum_m = jnp.concatenate([red[0:1, 0:H], red[1:2, 0:H]], axis=1)
    sum_q = jnp.concatenate([red[0:1, H:], red[1:2, H:]], axis=1)
    mu = sum_m * (1.0 / E)
    var = sum_q * (1.0 / E) - mu * mu
    m = jnp.concatenate([m0[...], m1[...]], axis=1)
    t = gamma[...] * (m - mu) * lax.rsqrt(var + 1e-5) + beta[...]
    y[...] = ef[...] + t * (1.0 / (1.0 + jnp.exp(-t)))


def _node_epi_body(acc0, acc1, cx, nf, gamma, beta, x):
    a0 = acc0[...]
    a1 = acc1[...]
    num = jnp.concatenate([a0[:, :H], a1[:, :H]], axis=1)
    den = jnp.concatenate([a0[:, H:], a1[:, H:]], axis=1)
    v = cx[...] + num / (den + 1e-6)
    mu = jnp.mean(v, axis=0, keepdims=True)
    var = jnp.mean(v * v, axis=0, keepdims=True) - mu * mu
    t = gamma[...] * (v - mu) * lax.rsqrt(var + 1e-5) + beta[...]
    x[...] = nf[...] + t * (1.0 / (1.0 + jnp.exp(-t)))


def kernel(node_feats, edge_feats, edge_index, W_src_gate, b_src_gate,
           W_dst_gate, b_dst_gate, W_edge_gate, b_edge_gate,
           W_dst_update, b_dst_update, W_src_update, b_src_update,
           bn_nodes_gamma, bn_nodes_beta, bn_edges_gamma, bn_edges_beta):
    src = edge_index[0]
    dst = edge_index[1]
    f32 = jnp.float32

    nb = 1000
    bdu = b_dst_update.reshape(1, D)
    bsu = b_src_update.reshape(1, D)
    ts0, ts1, e_dst, Cx = pl.pallas_call(
        _node_pre_body,
        grid=(N // nb,),
        in_specs=[
            pl.BlockSpec((nb, D), lambda i: (i, 0)),
            pl.BlockSpec((D, D), lambda i: (0, 0)),
            pl.BlockSpec((D, D), lambda i: (0, 0)),
            pl.BlockSpec((D, D), lambda i: (0, 0)),
            pl.BlockSpec((D, D), lambda i: (0, 0)),
            pl.BlockSpec((1, D), lambda i: (0, 0)),
            pl.BlockSpec((1, D), lambda i: (0, 0)),
        ],
        out_specs=[pl.BlockSpec((nb, D), lambda i: (i, 0))] * 4,
        out_shape=[jax.ShapeDtypeStruct((N, D), f32)] * 4,
    )(node_feats, W_src_gate, W_dst_gate, W_dst_update, W_src_update,
      bdu, bsu)

    eb = 4000
    gate_bias = (b_edge_gate + b_src_gate + b_dst_gate).reshape(1, D)
    eg0, eg1 = pl.pallas_call(
        _gate_body,
        grid=(E // eb,),
        in_specs=[
            pl.BlockSpec((eb, D), lambda i: (i, 0)),
            pl.BlockSpec((D, D), lambda i: (0, 0)),
            pl.BlockSpec((1, D), lambda i: (0, 0)),
        ],
        out_specs=[pl.BlockSpec((eb, H), lambda i: (i, 0))] * 2,
        out_shape=[jax.ShapeDtypeStruct((E, H), f32)] * 2,
    )(edge_feats, W_edge_gate, gate_bias)

    zer = jnp.zeros((N, D), f32)

    mesh = plsc.VectorSubcoreMesh(core_axis_name="c", subcore_axis_name="s")
    sc_fn = pl.kernel(
        _sc_body,
        out_type=[
            jax.ShapeDtypeStruct((E, H), f32),
            jax.ShapeDtypeStruct((E, H), f32),
            jax.ShapeDtypeStruct((N, D), f32),
            jax.ShapeDtypeStruct((N, D), f32),
            jax.ShapeDtypeStruct((2, NT, D), f32),
        ],
        mesh=mesh,
        scratch_types=(
            [pltpu.VMEM((C,), jnp.int32)] * 4 +
            [pltpu.VMEM((C, D), f32)] * 2 +
            [pltpu.VMEM((C, D), f32)] * 2 +
            [pltpu.VMEM((C, H), f32)] * 2 +
            [pltpu.VMEM((C, D), f32)] * 2 +
            [pltpu.VMEM((C,), jnp.int32)] * 2 +
            [pltpu.VMEM((D,), f32),
             pltpu.VMEM_SHARED((N, D), f32)] +
            [pltpu.SemaphoreType.DMA] * 14
        ),
    )
    m0, m1, acc0, acc1, stats = sc_fn(src, dst, ts0, ts1, e_dst,
                                      eg0, eg1, zer)

    ee = 8000
    y = pl.pallas_call(
        _edge_epi_body,
        grid=(E // ee,),
        in_specs=[
            pl.BlockSpec((ee, H), lambda i: (i, 0)),
            pl.BlockSpec((ee, H), lambda i: (i, 0)),
            pl.BlockSpec((ee, D), lambda i: (i, 0)),
            pl.BlockSpec((2, NT, D), lambda i: (0, 0, 0)),
            pl.BlockSpec((1, D), lambda i: (0, 0)),
            pl.BlockSpec((1, D), lambda i: (0, 0)),
        ],
        out_specs=pl.BlockSpec((ee, D), lambda i: (i, 0)),
        out_shape=jax.ShapeDtypeStruct((E, D), f32),
    )(m0, m1, edge_feats, stats, bn_edges_gamma.reshape(1, D),
      bn_edges_beta.reshape(1, D))

    x = pl.pallas_call(
        _node_epi_body,
        in_specs=[pl.BlockSpec((N, D), lambda: (0, 0))] * 4 +
                 [pl.BlockSpec((1, D), lambda: (0, 0))] * 2,
        out_specs=pl.BlockSpec((N, D), lambda: (0, 0)),
        out_shape=jax.ShapeDtypeStruct((N, D), f32),
    )(acc0, acc1, Cx, node_feats, bn_nodes_gamma.reshape(1, D),
      bn_nodes_beta.reshape(1, D))

    return (x, y)


# speedup vs baseline: 1.1530x; 1.1530x over previous
import functools

import jax
import jax.numpy as jnp
from jax import lax
from jax.experimental import pallas as pl
from jax.experimental.pallas import tpu as pltpu
from jax.experimental.pallas import tpu_sc as plsc

N = 10000
E = 320000
D = 128
H = 64
NT = 16
ET = E // NT
C = 32
NCHUNK = ET // C


def _node_pre_body(x_ref, wsg, wdg, wdu, wsu, bdu, bsu,
                   tsrc0, tsrc1, edst, cx):
    x = x_ref[...]
    t_es = x @ wsg[...]
    t_bh = x @ wdu[...] + bdu[...]
    tsrc0[...] = jnp.concatenate([t_es[:, :H], t_bh[:, :H]], axis=1)
    tsrc1[...] = jnp.concatenate([t_es[:, H:], t_bh[:, H:]], axis=1)
    edst[...] = x @ wdg[...]
    cx[...] = x @ wsu[...] + bsu[...]


def _gate_body(ef, weg, bias, eg0, eg1):
    t = ef[...] @ weg[...] + bias[...]
    eg0[...] = t[:, :H]
    eg1[...] = t[:, H:]


def _sc_body(src_hbm, dst_hbm, ts0, ts1, td, eg0, eg1, zer,
             m0_hbm, m1_hbm, acc0_hbm, acc1_hbm, stats_hbm,
             sq0, sq1, tq0, tq1, av0, av1, dv0, dv1, ev0, ev1, cb0, cb1,
             lo0, lo1, stat_v, acc_sh,
             ss0, ss1, st0, st1, sa0, sa1, sd0, sd1, se0, se1,
             sm0, sm1, sx0, sx1):
    c = lax.axis_index("c")
    s = lax.axis_index("s")
    srcq = [sq0, sq1]
    dstq = [tq0, tq1]
    av = [av0, av1]
    dvv = [dv0, dv1]
    ev = [ev0, ev1]
    cbv = [cb0, cb1]
    lov = [lo0, lo1]
    sem_s = [ss0, ss1]
    sem_t = [st0, st1]
    sem_a = [sa0, sa1]
    sem_d = [sd0, sd1]
    sem_e = [se0, se1]
    sem_m = [sm0, sm1]
    sem_x = [sx0, sx1]

    @pl.when(s == 0)
    def _():
        pltpu.sync_copy(zer, acc_sh)

    plsc.subcore_barrier()

    def run_half(ts, eg, off, m_hbm):
        tile_base = s * ET

        def cbase(ci):
            return tile_base + jnp.minimum(ci * C, ET - C)

        def idx_issue(ci, q):
            b = cbase(ci)
            pltpu.async_copy(src_hbm.at[pl.ds(b, C)], srcq[q], sem_s[q])
            pltpu.async_copy(dst_hbm.at[pl.ds(b, C)], dstq[q], sem_t[q])

        def idx_wait(q):
            pltpu.make_async_copy(src_hbm.at[pl.ds(0, C)], srcq[q],
                                  sem_s[q]).wait()
            pltpu.make_async_copy(dst_hbm.at[pl.ds(0, C)], dstq[q],
                                  sem_t[q]).wait()

        def gather_issue(ci, g, q):
            b = cbase(ci)
            pltpu.async_copy(ts.at[srcq[q]], av[g], sem_a[g])
            pltpu.async_copy(td.at[dstq[q]], dvv[g], sem_d[g])
            pltpu.async_copy(eg.at[pl.ds(b, C)], ev[g], sem_e[g])

        def gather_wait(g, q):
            pltpu.make_async_copy(ts.at[srcq[q]], av[g], sem_a[g]).wait()
            pltpu.make_async_copy(td.at[dstq[q]], dvv[g], sem_d[g]).wait()
            pltpu.make_async_copy(eg.at[pl.ds(0, C)], ev[g], sem_e[g]).wait()

        def ilo_compute(g, q):
            for j in range(C // 16):
                sl = pl.ds(j * 16, 16)
                lov[g][sl] = dstq[q][sl]

        def rows(g, carry):
            def row_body(r2, rc):
                acc = list(rc)
                for p in range(2):
                    r = r2 * 2 + p
                    for k in range(4):
                        sl = pl.ds(k * 16, 16)
                        slh = pl.ds(off + k * 16, 16)
                        m = av[g][r, sl] + dvv[g][r, slh] + ev[g][r, sl]
                        ev[g][r, sl] = m
                        sig = 1.0 / (1.0 + jnp.exp(-m))
                        cbv[g][r, sl] = sig * av[g][r, pl.ds(H + k * 16, 16)]
                        cbv[g][r, pl.ds(H + k * 16, 16)] = sig
                        acc[k] = acc[k] + m
                        acc[4 + k] = acc[4 + k] + m * m
                return tuple(acc)

            return lax.fori_loop(0, C // 2, row_body, carry)

        def outs_issue(ci, g):
            b = cbase(ci)
            pltpu.async_copy(ev[g], m_hbm.at[pl.ds(b, C)], sem_m[g])
            pltpu.async_copy(cbv[g], acc_sh.at[lov[g]], sem_x[g], add=True)

        def outs_wait(g):
            pltpu.make_async_copy(ev[g], m_hbm.at[pl.ds(0, C)],
                                  sem_m[g]).wait()
            pltpu.make_async_copy(cbv[g], acc_sh.at[lov[g]], sem_x[g]).wait()

        idx_issue(0, 0)
        idx_wait(0)
        ilo_compute(1, 0)
        gather_issue(0, 1, 0)
        idx_issue(1, 1)
        gather_wait(1, 0)
        idx_wait(1)
        gather_issue(1, 0, 1)
        ilo_compute(0, 1)
        idx_issue(2, 0)
        z = jnp.zeros((16,), jnp.float32)
        carry = rows(1, (z,) * 8)
        outs_issue(0, 1)

        def pair_body(i2, carry):
            i = 1 + 2 * i2
            gather_wait(0, 1)
            outs_wait(1)
            idx_wait(0)
            gather_issue(i + 1, 1, 0)
            ilo_compute(1, 0)
            idx_issue(i + 2, 1)
            carry = rows(0, carry)
            outs_issue(i, 0)
            gather_wait(1, 0)
            outs_wait(0)
            idx_wait(1)
            gather_issue(i + 2, 0, 1)
            ilo_compute(0, 1)
            idx_issue(i + 3, 0)
            carry = rows(1, carry)
            outs_issue(i + 1, 1)
            return carry

        carry = lax.fori_loop(0, (NCHUNK - 1) // 2, pair_body, carry)

        outs_wait(1)
        gather_wait(0, 1)
        idx_wait(0)

        for k in range(4):
            stat_v[pl.ds(k * 16, 16)] = carry[k]
            stat_v[pl.ds(H + k * 16, 16)] = carry[4 + k]
        pltpu.sync_copy(stat_v, stats_hbm.at[c, s])

    @pl.when(c == 0)
    def _():
        run_half(ts0, eg0, 0, m0_hbm)

    @pl.when(c == 1)
    def _():
        run_half(ts1, eg1, H, m1_hbm)

    plsc.subcore_barrier()

    @pl.when((s == 0) & (c == 0))
    def _():
        pltpu.sync_copy(acc_sh, acc0_hbm)

    @pl.when((s == 0) & (c == 1))
    def _():
        pltpu.sync_copy(acc_sh, acc1_hbm)


def _edge_epi_body(m0, m1, ef, stats, gamma, beta, y):
    st = stats[...]
    red = jnp.sum(st, axis=1)
    sum_m = jnp.concatenate([red[0:1, 0:H], red[1:2, 0:H]], axis=1)
    sum_q = jnp.concatenate([red[0:1, H:], red[1:2, H:]], axis=1)
    mu = sum_m * (1.0 / E)
    var = sum_q * (1.0 / E) - mu * mu
    m = jnp.concatenate([m0[...], m1[...]], axis=1)
    t = gamma[...] * (m - mu) * lax.rsqrt(var + 1e-5) + beta[...]
    y[...] = ef[...] + t * (1.0 / (1.0 + jnp.exp(-t)))


def _node_epi_body(acc0, acc1, cx, nf, gamma, beta, x):
    a0 = acc0[...]
    a1 = acc1[...]
    num = jnp.concatenate([a0[:, :H], a1[:, :H]], axis=1)
    den = jnp.concatenate([a0[:, H:], a1[:, H:]], axis=1)
    v = cx[...] + num / (den + 1e-6)
    mu = jnp.mean(v, axis=0, keepdims=True)
    var = jnp.mean(v * v, axis=0, keepdims=True) - mu * mu
    t = gamma[...] * (v - mu) * lax.rsqrt(var + 1e-5) + beta[...]
    x[...] = nf[...] + t * (1.0 / (1.0 + jnp.exp(-t)))


def kernel(node_feats, edge_feats, edge_index, W_src_gate, b_src_gate,
           W_dst_gate, b_dst_gate, W_edge_gate, b_edge_gate,
           W_dst_update, b_dst_update, W_src_update, b_src_update,
           bn_nodes_gamma, bn_nodes_beta, bn_edges_gamma, bn_edges_beta):
    src = edge_index[0]
    dst = edge_index[1]
    f32 = jnp.float32

    nb = 1000
    bdu = b_dst_update.reshape(1, D)
    bsu = b_src_update.reshape(1, D)
    ts0, ts1, e_dst, Cx = pl.pallas_call(
        _node_pre_body,
        grid=(N // nb,),
        in_specs=[
            pl.BlockSpec((nb, D), lambda i: (i, 0)),
            pl.BlockSpec((D, D), lambda i: (0, 0)),
            pl.BlockSpec((D, D), lambda i: (0, 0)),
            pl.BlockSpec((D, D), lambda i: (0, 0)),
            pl.BlockSpec((D, D), lambda i: (0, 0)),
            pl.BlockSpec((1, D), lambda i: (0, 0)),
            pl.BlockSpec((1, D), lambda i: (0, 0)),
        ],
        out_specs=[pl.BlockSpec((nb, D), lambda i: (i, 0))] * 4,
        out_shape=[jax.ShapeDtypeStruct((N, D), f32)] * 4,
    )(node_feats, W_src_gate, W_dst_gate, W_dst_update, W_src_update,
      bdu, bsu)

    eb = 4000
    gate_bias = (b_edge_gate + b_src_gate + b_dst_gate).reshape(1, D)
    eg0, eg1 = pl.pallas_call(
        _gate_body,
        grid=(E // eb,),
        in_specs=[
            pl.BlockSpec((eb, D), lambda i: (i, 0)),
            pl.BlockSpec((D, D), lambda i: (0, 0)),
            pl.BlockSpec((1, D), lambda i: (0, 0)),
        ],
        out_specs=[pl.BlockSpec((eb, H), lambda i: (i, 0))] * 2,
        out_shape=[jax.ShapeDtypeStruct((E, H), f32)] * 2,
    )(edge_feats, W_edge_gate, gate_bias)

    zer = jnp.zeros((N, D), f32)

    mesh = plsc.VectorSubcoreMesh(core_axis_name="c", subcore_axis_name="s")
    sc_fn = pl.kernel(
        _sc_body,
        out_type=[
            jax.ShapeDtypeStruct((E, H), f32),
            jax.ShapeDtypeStruct((E, H), f32),
            jax.ShapeDtypeStruct((N, D), f32),
            jax.ShapeDtypeStruct((N, D), f32),
            jax.ShapeDtypeStruct((2, NT, D), f32),
        ],
        mesh=mesh,
        scratch_types=(
            [pltpu.VMEM((C,), jnp.int32)] * 4 +
            [pltpu.VMEM((C, D), f32)] * 2 +
            [pltpu.VMEM((C, D), f32)] * 2 +
            [pltpu.VMEM((C, H), f32)] * 2 +
            [pltpu.VMEM((C, D), f32)] * 2 +
            [pltpu.VMEM((C,), jnp.int32)] * 2 +
            [pltpu.VMEM((D,), f32),
             pltpu.VMEM_SHARED((N, D), f32)] +
            [pltpu.SemaphoreType.DMA] * 14
        ),
    )
    m0, m1, acc0, acc1, stats = sc_fn(src, dst, ts0, ts1, e_dst,
                                      eg0, eg1, zer)

    ee = 8000
    y = pl.pallas_call(
        _edge_epi_body,
        grid=(E // ee,),
        in_specs=[
            pl.BlockSpec((ee, H), lambda i: (i, 0)),
            pl.BlockSpec((ee, H), lambda i: (i, 0)),
            pl.BlockSpec((ee, D), lambda i: (i, 0)),
            pl.BlockSpec((2, NT, D), lambda i: (0, 0, 0)),
            pl.BlockSpec((1, D), lambda i: (0, 0)),
            pl.BlockSpec((1, D), lambda i: (0, 0)),
        ],
        out_specs=pl.BlockSpec((ee, D), lambda i: (i, 0)),
        out_shape=jax.ShapeDtypeStruct((E, D), f32),
    )(m0, m1, edge_feats, stats, bn_edges_gamma.reshape(1, D),
      bn_edges_beta.reshape(1, D))

    x = pl.pallas_call(
        _node_epi_body,
        in_specs=[pl.BlockSpec((N, D), lambda: (0, 0))] * 4 +
                 [pl.BlockSpec((1, D), lambda: (0, 0))] * 2,
        out_specs=pl.BlockSpec((N, D), lambda: (0, 0)),
        out_shape=jax.ShapeDtypeStruct((N, D), f32),
    )(acc0, acc1, Cx, node_feats, bn_nodes_gamma.reshape(1, D),
      bn_nodes_beta.reshape(1, D))

    return (x, y)


# speedup vs baseline: 1.4133x; 1.2258x over previous
import functools

import jax
import jax.numpy as jnp
from jax import lax
from jax.experimental import pallas as pl
from jax.experimental.pallas import tpu as pltpu
from jax.experimental.pallas import tpu_sc as plsc

N = 10000
E = 320000
D = 128
H = 64
NT = 16
ET = E // NT
C = 32
NCHUNK = ET // C


def _node_pre_body(x_ref, wsg, wdg, wdu, wsu, bdu, bsu,
                   tsrc0, tsrc1, edst, cx):
    x = x_ref[...]
    t_es = x @ wsg[...]
    t_bh = x @ wdu[...] + bdu[...]
    tsrc0[...] = jnp.concatenate([t_es[:, :H], t_bh[:, :H]], axis=1)
    tsrc1[...] = jnp.concatenate([t_es[:, H:], t_bh[:, H:]], axis=1)
    edst[...] = x @ wdg[...]
    cx[...] = x @ wsu[...] + bsu[...]


def _gate_body(ef, weg, bias, eg0, eg1):
    t = ef[...] @ weg[...] + bias[...]
    eg0[...] = t[:, :H]
    eg1[...] = t[:, H:]


def _sc_body(src_hbm, dst_hbm, ts0, ts1, td, eg0, eg1, zer,
             m0_hbm, m1_hbm, acc0_hbm, acc1_hbm, stats_hbm,
             sq0, sq1, sq2, tq0, tq1, tq2, av0, av1, av2, dv0, dv1, dv2,
             ev0, ev1, ev2, cb0, cb1, lo0, lo1, lo2, stat_v, acc_sh,
             ss0, ss1, ss2, st0, st1, st2, sa0, sa1, sa2, sd0, sd1, sd2,
             se0, se1, se2, sm0, sm1, sx0, sx1):
    c = lax.axis_index("c")
    s = lax.axis_index("s")
    srcq = [sq0, sq1, sq2]
    dstq = [tq0, tq1, tq2]
    av = [av0, av1, av2]
    dvv = [dv0, dv1, dv2]
    ev = [ev0, ev1, ev2]
    cbv = [cb0, cb1]
    lov = [lo0, lo1, lo2]
    sem_s = [ss0, ss1, ss2]
    sem_t = [st0, st1, st2]
    sem_a = [sa0, sa1, sa2]
    sem_d = [sd0, sd1, sd2]
    sem_e = [se0, se1, se2]
    sem_m = [sm0, sm1]
    sem_x = [sx0, sx1]

    @pl.when(s == 0)
    def _():
        pltpu.sync_copy(zer, acc_sh)

    plsc.subcore_barrier()

    def run_half(ts, eg, off, m_hbm):
        tile_base = s * ET

        def cbase(ci):
            return tile_base + jnp.minimum(ci * C, ET - C)

        def idx_issue(ci, q):
            b = cbase(ci)
            pltpu.async_copy(src_hbm.at[pl.ds(b, C)], srcq[q], sem_s[q])
            pltpu.async_copy(dst_hbm.at[pl.ds(b, C)], dstq[q], sem_t[q])

        def idx_wait(q):
            pltpu.make_async_copy(src_hbm.at[pl.ds(0, C)], srcq[q],
                                  sem_s[q]).wait()
            pltpu.make_async_copy(dst_hbm.at[pl.ds(0, C)], dstq[q],
                                  sem_t[q]).wait()

        def gather_issue(ci, g, q):
            b = cbase(ci)
            pltpu.async_copy(ts.at[srcq[q]], av[g], sem_a[g])
            pltpu.async_copy(td.at[dstq[q]], dvv[g], sem_d[g])
            pltpu.async_copy(eg.at[pl.ds(b, C)], ev[g], sem_e[g])

        def gather_wait(g, q):
            pltpu.make_async_copy(ts.at[srcq[q]], av[g], sem_a[g]).wait()
            pltpu.make_async_copy(td.at[dstq[q]], dvv[g], sem_d[g]).wait()
            pltpu.make_async_copy(eg.at[pl.ds(0, C)], ev[g], sem_e[g]).wait()

        def ilo_compute(g):
            for j in range(C // 16):
                sl = pl.ds(j * 16, 16)
                lov[g][sl] = dstq[g][sl]

        def rows(g, cb, carry):
            def row_body(r2, rc):
                acc = list(rc)
                for p in range(2):
                    r = r2 * 2 + p
                    for k in range(4):
                        sl = pl.ds(k * 16, 16)
                        slh = pl.ds(off + k * 16, 16)
                        m = av[g][r, sl] + dvv[g][r, slh] + ev[g][r, sl]
                        ev[g][r, sl] = m
                        sig = 1.0 / (1.0 + jnp.exp(-m))
                        cbv[cb][r, sl] = sig * av[g][r,
                                                    pl.ds(H + k * 16, 16)]
                        cbv[cb][r, pl.ds(H + k * 16, 16)] = sig
                        acc[k] = acc[k] + m
                        acc[4 + k] = acc[4 + k] + m * m
                return tuple(acc)

            return lax.fori_loop(0, C // 2, row_body, carry)

        def outs_issue(ci, g, cb):
            b = cbase(ci)
            pltpu.async_copy(ev[g], m_hbm.at[pl.ds(b, C)], sem_m[cb])
            pltpu.async_copy(cbv[cb], acc_sh.at[lov[g]], sem_x[cb],
                             add=True)

        def outs_wait(g, cb):
            pltpu.make_async_copy(ev[g], m_hbm.at[pl.ds(0, C)],
                                  sem_m[cb]).wait()
            pltpu.make_async_copy(cbv[cb], acc_sh.at[lov[g]],
                                  sem_x[cb]).wait()

        z = jnp.zeros((16,), jnp.float32)

        idx_issue(0, 0)
        idx_issue(1, 1)
        idx_wait(0)
        ilo_compute(0)
        gather_issue(0, 0, 0)
        idx_wait(1)
        ilo_compute(1)
        gather_issue(1, 1, 1)
        idx_issue(2, 2)
        gather_wait(0, 0)
        idx_wait(2)
        gather_issue(2, 2, 2)
        ilo_compute(2)
        idx_issue(3, 0)
        carry = rows(0, 0, (z,) * 8)
        outs_issue(0, 0, 0)

        def six_body(t, carry):
            i0 = 1 + 6 * t
            for p in range(6):
                i = i0 + p
                g3 = (1 + p) % 3
                cb2 = (1 + p) % 2
                q3 = p % 3
                gather_wait(g3, g3)
                outs_wait(p % 3, p % 2)
                idx_wait(q3)
                gather_issue(i + 2, q3, q3)
                ilo_compute(q3)
                idx_issue(i + 3, g3)
                carry = rows(g3, cb2, carry)
                outs_issue(i, g3, cb2)
            return carry

        carry = lax.fori_loop(0, (NCHUNK - 1) // 6, six_body, carry)

        outs_wait(0, 0)
        gather_wait(1, 1)
        gather_wait(2, 2)
        idx_wait(0)

        for k in range(4):
            stat_v[pl.ds(k * 16, 16)] = carry[k]
            stat_v[pl.ds(H + k * 16, 16)] = carry[4 + k]
        pltpu.sync_copy(stat_v, stats_hbm.at[c, s])

    @pl.when(c == 0)
    def _():
        run_half(ts0, eg0, 0, m0_hbm)

    @pl.when(c == 1)
    def _():
        run_half(ts1, eg1, H, m1_hbm)

    plsc.subcore_barrier()

    @pl.when((s == 0) & (c == 0))
    def _():
        pltpu.sync_copy(acc_sh, acc0_hbm)

    @pl.when((s == 0) & (c == 1))
    def _():
        pltpu.sync_copy(acc_sh, acc1_hbm)


def _edge_epi_body(m0, m1, ef, stats, gamma, beta, y):
    st = stats[...]
    red = jnp.sum(st, axis=1)
    sum_m = jnp.concatenate([red[0:1, 0:H], red[1:2, 0:H]], axis=1)
    sum_q = jnp.concatenate([red[0:1, H:], red[1:2, H:]], axis=1)
    mu = sum_m * (1.0 / E)
    var = sum_q * (1.0 / E) - mu * mu
    m = jnp.concatenate([m0[...], m1[...]], axis=1)
    t = gamma[...] * (m - mu) * lax.rsqrt(var + 1e-5) + beta[...]
    y[...] = ef[...] + t * (1.0 / (1.0 + jnp.exp(-t)))


def _node_epi_body(acc0, acc1, cx, nf, gamma, beta, x):
    a0 = acc0[...]
    a1 = acc1[...]
    num = jnp.concatenate([a0[:, :H], a1[:, :H]], axis=1)
    den = jnp.concatenate([a0[:, H:], a1[:, H:]], axis=1)
    v = cx[...] + num / (den + 1e-6)
    mu = jnp.mean(v, axis=0, keepdims=True)
    var = jnp.mean(v * v, axis=0, keepdims=True) - mu * mu
    t = gamma[...] * (v - mu) * lax.rsqrt(var + 1e-5) + beta[...]
    x[...] = nf[...] + t * (1.0 / (1.0 + jnp.exp(-t)))


def kernel(node_feats, edge_feats, edge_index, W_src_gate, b_src_gate,
           W_dst_gate, b_dst_gate, W_edge_gate, b_edge_gate,
           W_dst_update, b_dst_update, W_src_update, b_src_update,
           bn_nodes_gamma, bn_nodes_beta, bn_edges_gamma, bn_edges_beta):
    src = edge_index[0]
    dst = edge_index[1]
    f32 = jnp.float32

    nb = 1000
    bdu = b_dst_update.reshape(1, D)
    bsu = b_src_update.reshape(1, D)
    ts0, ts1, e_dst, Cx = pl.pallas_call(
        _node_pre_body,
        grid=(N // nb,),
        in_specs=[
            pl.BlockSpec((nb, D), lambda i: (i, 0)),
            pl.BlockSpec((D, D), lambda i: (0, 0)),
            pl.BlockSpec((D, D), lambda i: (0, 0)),
            pl.BlockSpec((D, D), lambda i: (0, 0)),
            pl.BlockSpec((D, D), lambda i: (0, 0)),
            pl.BlockSpec((1, D), lambda i: (0, 0)),
            pl.BlockSpec((1, D), lambda i: (0, 0)),
        ],
        out_specs=[pl.BlockSpec((nb, D), lambda i: (i, 0))] * 4,
        out_shape=[jax.ShapeDtypeStruct((N, D), f32)] * 4,
    )(node_feats, W_src_gate, W_dst_gate, W_dst_update, W_src_update,
      bdu, bsu)

    eb = 4000
    gate_bias = (b_edge_gate + b_src_gate + b_dst_gate).reshape(1, D)
    eg0, eg1 = pl.pallas_call(
        _gate_body,
        grid=(E // eb,),
        in_specs=[
            pl.BlockSpec((eb, D), lambda i: (i, 0)),
            pl.BlockSpec((D, D), lambda i: (0, 0)),
            pl.BlockSpec((1, D), lambda i: (0, 0)),
        ],
        out_specs=[pl.BlockSpec((eb, H), lambda i: (i, 0))] * 2,
        out_shape=[jax.ShapeDtypeStruct((E, H), f32)] * 2,
    )(edge_feats, W_edge_gate, gate_bias)

    zer = jnp.zeros((N, D), f32)

    mesh = plsc.VectorSubcoreMesh(core_axis_name="c", subcore_axis_name="s")
    sc_fn = pl.kernel(
        _sc_body,
        out_type=[
            jax.ShapeDtypeStruct((E, H), f32),
            jax.ShapeDtypeStruct((E, H), f32),
            jax.ShapeDtypeStruct((N, D), f32),
            jax.ShapeDtypeStruct((N, D), f32),
            jax.ShapeDtypeStruct((2, NT, D), f32),
        ],
        mesh=mesh,
        scratch_types=(
            [pltpu.VMEM((C,), jnp.int32)] * 6 +
            [pltpu.VMEM((C, D), f32)] * 3 +
            [pltpu.VMEM((C, D), f32)] * 3 +
            [pltpu.VMEM((C, H), f32)] * 3 +
            [pltpu.VMEM((C, D), f32)] * 2 +
            [pltpu.VMEM((C,), jnp.int32)] * 3 +
            [pltpu.VMEM((D,), f32),
             pltpu.VMEM_SHARED((N, D), f32)] +
            [pltpu.SemaphoreType.DMA] * 19
        ),
    )
    m0, m1, acc0, acc1, stats = sc_fn(src, dst, ts0, ts1, e_dst,
                                      eg0, eg1, zer)

    ee = 8000
    y = pl.pallas_call(
        _edge_epi_body,
        grid=(E // ee,),
        in_specs=[
            pl.BlockSpec((ee, H), lambda i: (i, 0)),
            pl.BlockSpec((ee, H), lambda i: (i, 0)),
            pl.BlockSpec((ee, D), lambda i: (i, 0)),
            pl.BlockSpec((2, NT, D), lambda i: (0, 0, 0)),
            pl.BlockSpec((1, D), lambda i: (0, 0)),
            pl.BlockSpec((1, D), lambda i: (0, 0)),
        ],
        out_specs=pl.BlockSpec((ee, D), lambda i: (i, 0)),
        out_shape=jax.ShapeDtypeStruct((E, D), f32),
    )(m0, m1, edge_feats, stats, bn_edges_gamma.reshape(1, D),
      bn_edges_beta.reshape(1, D))

    x = pl.pallas_call(
        _node_epi_body,
        in_specs=[pl.BlockSpec((N, D), lambda: (0, 0))] * 4 +
                 [pl.BlockSpec((1, D), lambda: (0, 0))] * 2,
        out_specs=pl.BlockSpec((N, D), lambda: (0, 0)),
        out_shape=jax.ShapeDtypeStruct((N, D), f32),
    )(acc0, acc1, Cx, node_feats, bn_nodes_gamma.reshape(1, D),
      bn_nodes_beta.reshape(1, D))

    return (x, y)
